# Initial kernel scaffold; baseline (speedup 1.0000x reference)
import functools

import jax
import jax.numpy as jnp
from jax import lax
from jax.experimental import pallas as pl
from jax.experimental.pallas import tpu as pltpu
from jax.experimental.pallas import tpu_sc as plsc

N_NODES = 10000
N_EDGES = 320000
D = 128

NC = 2
NS = 16
NW = NC * NS
E_PER_W = N_EDGES // NW
CHUNK = 80
N_CHUNKS = E_PER_W // CHUNK
ROWS_PER_TILE = N_NODES // NS


def _sc_scatter_sum(features, src, dst, zeros):
    mesh = plsc.VectorSubcoreMesh(
        core_axis_name="c", subcore_axis_name="s", num_cores=NC, num_subcores=NS
    )

    @functools.partial(
        pl.kernel,
        out_type=jax.ShapeDtypeStruct((NC, N_NODES, D), jnp.float32),
        mesh=mesh,
        scratch_types=[
            pltpu.VMEM_SHARED((N_NODES, D), jnp.float32),
            pltpu.VMEM((CHUNK,), jnp.int32),
            pltpu.VMEM((CHUNK,), jnp.int32),
            pltpu.VMEM((CHUNK, D), jnp.float32),
            pltpu.SemaphoreType.DMA,
        ],
    )
    def k(feat_hbm, src_hbm, dst_hbm, zeros_hbm, out_hbm, acc, src_v, dst_v, rows_v, sem):
        c = lax.axis_index("c")
        s = lax.axis_index("s")
        wid = s * NC + c

        pltpu.sync_copy(zeros_hbm, acc.at[pl.ds(s * ROWS_PER_TILE, ROWS_PER_TILE)])
        plsc.subcore_barrier()

        base = wid * E_PER_W

        def body(i, _):
            off = base + i * CHUNK
            pltpu.sync_copy(src_hbm.at[pl.ds(off, CHUNK)], src_v)
            pltpu.sync_copy(dst_hbm.at[pl.ds(off, CHUNK)], dst_v)
            pltpu.async_copy(feat_hbm.at[src_v], rows_v, sem).wait()
            pltpu.sync_copy(rows_v, acc.at[dst_v], add=True)
            return ()

        lax.fori_loop(0, N_CHUNKS, body, ())

        plsc.subcore_barrier()
        pltpu.sync_copy(
            acc.at[pl.ds(s * ROWS_PER_TILE, ROWS_PER_TILE)],
            out_hbm.at[c, pl.ds(s * ROWS_PER_TILE, ROWS_PER_TILE)],
        )

    return k(features, src, dst, zeros)


def _tc_body(f_ref, p0_ref, p1_ref, wt_ref, b_ref, t_ref, o_ref):
    t1 = t_ref[0, 0]
    t2 = t_ref[0, 1]
    wt = wt_ref[...]
    b = b_ref[...]
    a1 = jnp.dot(f_ref[...], wt, preferred_element_type=jnp.float32) + b
    hn = p0_ref[...] + p1_ref[...]
    a2 = jnp.dot(hn, wt, preferred_element_type=jnp.float32) + b
    o_ref[...] = t1 * jnp.maximum(a1, 0.0) + t2 * jnp.maximum(a2, 0.0)


def _tc_combine(features, partials, W, b, theta1, theta2):
    wt = W.T
    b2 = b.reshape(1, D)
    thetas = jnp.stack([theta1[0], theta2[0]]).reshape(1, 2)
    R = 1000
    grid = (N_NODES // R,)
    return pl.pallas_call(
        _tc_body,
        grid=grid,
        in_specs=[
            pl.BlockSpec((R, D), lambda i: (i, 0)),
            pl.BlockSpec((R, D), lambda i: (i, 0)),
            pl.BlockSpec((R, D), lambda i: (i, 0)),
            pl.BlockSpec((D, D), lambda i: (0, 0)),
            pl.BlockSpec((1, D), lambda i: (0, 0)),
            pl.BlockSpec(memory_space=pltpu.SMEM),
        ],
        out_specs=pl.BlockSpec((R, D), lambda i: (i, 0)),
        out_shape=jax.ShapeDtypeStruct((N_NODES, D), jnp.float32),
    )(features, partials[0], partials[1], wt, b2, thetas)


@jax.jit
def kernel(features, edge_index, W, b, theta1, theta2):
    src = edge_index[0].astype(jnp.int32)
    dst = edge_index[1].astype(jnp.int32)
    zeros = jnp.zeros((ROWS_PER_TILE, D), jnp.float32)
    partials = _sc_scatter_sum(features, src, dst, zeros)
    return _tc_combine(features, partials, W, b, theta1, theta2)


# speedup vs baseline: 5.3466x; 5.3466x over previous
import functools

import jax
import jax.numpy as jnp
from jax import lax
from jax.experimental import pallas as pl
from jax.experimental.pallas import tpu as pltpu
from jax.experimental.pallas import tpu_sc as plsc

N_NODES = 10000
N_EDGES = 320000
D = 128

NC = 2
NS = 16
NW = NC * NS
E_PER_W = N_EDGES // NW
CHUNK = 80
N_CHUNKS = E_PER_W // CHUNK
ROWS_PER_TILE = 624
TAIL_OFF = ROWS_PER_TILE * NS
TAIL_ROWS = N_NODES - TAIL_OFF


def _sc_scatter_sum(features, src, dst, zeros):
    mesh = plsc.VectorSubcoreMesh(
        core_axis_name="c", subcore_axis_name="s", num_cores=NC, num_subcores=NS
    )

    @functools.partial(
        pl.kernel,
        out_type=jax.ShapeDtypeStruct((NC, N_NODES, D), jnp.float32),
        mesh=mesh,
        scratch_types=[
            pltpu.VMEM_SHARED((N_NODES, D), jnp.float32),
            pltpu.VMEM((CHUNK,), jnp.int32),
            pltpu.VMEM((CHUNK,), jnp.int32),
            pltpu.VMEM((CHUNK, D), jnp.float32),
            pltpu.SemaphoreType.DMA,
        ],
    )
    def k(feat_hbm, src_hbm, dst_hbm, zeros_hbm, out_hbm, acc, src_v, dst_v, rows_v, sem):
        c = lax.axis_index("c")
        s = lax.axis_index("s")
        wid = s * NC + c

        pltpu.sync_copy(zeros_hbm, acc.at[pl.ds(s * ROWS_PER_TILE, ROWS_PER_TILE)])

        @pl.when(s == NS - 1)
        def _():
            pltpu.sync_copy(zeros_hbm.at[pl.ds(0, TAIL_ROWS)],
                            acc.at[pl.ds(TAIL_OFF, TAIL_ROWS)])

        plsc.subcore_barrier()

        base = wid * E_PER_W

        def body(i, _):
            off = base + i * CHUNK
            pltpu.sync_copy(src_hbm.at[pl.ds(off, CHUNK)], src_v)
            pltpu.sync_copy(dst_hbm.at[pl.ds(off, CHUNK)], dst_v)
            pltpu.async_copy(feat_hbm.at[src_v], rows_v, sem).wait()
            pltpu.sync_copy(rows_v, acc.at[dst_v], add=True)
            return ()

        lax.fori_loop(0, N_CHUNKS, body, ())

        plsc.subcore_barrier()
        pltpu.sync_copy(
            acc.at[pl.ds(s * ROWS_PER_TILE, ROWS_PER_TILE)],
            out_hbm.at[c, pl.ds(s * ROWS_PER_TILE, ROWS_PER_TILE)],
        )

        @pl.when(s == NS - 1)
        def _():
            pltpu.sync_copy(acc.at[pl.ds(TAIL_OFF, TAIL_ROWS)],
                            out_hbm.at[c, pl.ds(TAIL_OFF, TAIL_ROWS)])

    return k(features, src, dst, zeros)


def _tc_body(f_ref, p0_ref, p1_ref, wt_ref, b_ref, t_ref, o_ref):
    t1 = t_ref[0, 0]
    t2 = t_ref[0, 1]
    wt = wt_ref[...]
    b = b_ref[...]
    a1 = jnp.dot(f_ref[...], wt, preferred_element_type=jnp.float32) + b
    hn = p0_ref[...] + p1_ref[...]
    a2 = jnp.dot(hn, wt, preferred_element_type=jnp.float32) + b
    o_ref[...] = t1 * jnp.maximum(a1, 0.0) + t2 * jnp.maximum(a2, 0.0)


def _tc_combine(features, partials, W, b, theta1, theta2):
    wt = W.T
    b2 = b.reshape(1, D)
    thetas = jnp.stack([theta1[0], theta2[0]]).reshape(1, 2)
    R = 1000
    grid = (N_NODES // R,)
    return pl.pallas_call(
        _tc_body,
        grid=grid,
        in_specs=[
            pl.BlockSpec((R, D), lambda i: (i, 0)),
            pl.BlockSpec((R, D), lambda i: (i, 0)),
            pl.BlockSpec((R, D), lambda i: (i, 0)),
            pl.BlockSpec((D, D), lambda i: (0, 0)),
            pl.BlockSpec((1, D), lambda i: (0, 0)),
            pl.BlockSpec(memory_space=pltpu.SMEM),
        ],
        out_specs=pl.BlockSpec((R, D), lambda i: (i, 0)),
        out_shape=jax.ShapeDtypeStruct((N_NODES, D), jnp.float32),
    )(features, partials[0], partials[1], wt, b2, thetas)


@jax.jit
def kernel(features, edge_index, W, b, theta1, theta2):
    src = edge_index[0].astype(jnp.int32)
    dst = edge_index[1].astype(jnp.int32)
    zeros = jnp.zeros((ROWS_PER_TILE, D), jnp.float32)
    partials = _sc_scatter_sum(features, src, dst, zeros)
    return _tc_combine(features, partials, W, b, theta1, theta2)


# speedup vs baseline: 9.6686x; 1.8084x over previous
import functools

import jax
import jax.numpy as jnp
from jax import lax
from jax.experimental import pallas as pl
from jax.experimental.pallas import tpu as pltpu
from jax.experimental.pallas import tpu_sc as plsc

N_NODES = 10000
N_EDGES = 320000
D = 128

NC = 2
NS = 16
NW = NC * NS
E_PER_W = N_EDGES // NW
CHUNK = 128
FULL_CHUNKS = E_PER_W // CHUNK
TAIL_E = E_PER_W - FULL_CHUNKS * CHUNK
ROWS_PER_TILE = 624
TAIL_OFF = ROWS_PER_TILE * NS
TAIL_ROWS = N_NODES - TAIL_OFF


def _sc_scatter_sum(features, src, dst, zeros):
    mesh = plsc.VectorSubcoreMesh(
        core_axis_name="c", subcore_axis_name="s", num_cores=NC, num_subcores=NS
    )

    @functools.partial(
        pl.kernel,
        out_type=jax.ShapeDtypeStruct((NC, N_NODES, D), jnp.float32),
        mesh=mesh,
        scratch_types=[
            pltpu.VMEM_SHARED((N_NODES, D), jnp.float32),
            pltpu.VMEM((CHUNK,), jnp.int32),
            pltpu.VMEM((CHUNK,), jnp.int32),
            pltpu.VMEM((CHUNK,), jnp.int32),
            pltpu.VMEM((CHUNK,), jnp.int32),
            pltpu.VMEM((TAIL_E,), jnp.int32),
            pltpu.VMEM((TAIL_E,), jnp.int32),
            pltpu.VMEM((CHUNK, D), jnp.float32),
            pltpu.VMEM((CHUNK, D), jnp.float32),
            pltpu.VMEM((TAIL_E, D), jnp.float32),
            pltpu.SemaphoreType.DMA,
            pltpu.SemaphoreType.DMA,
        ],
    )
    def k(feat_hbm, src_hbm, dst_hbm, zeros_hbm, out_hbm,
          acc, src0, src1, dst0, dst1, srct, dstt, rows0, rows1, rowst,
          sem0, sem1):
        c = lax.axis_index("c")
        s = lax.axis_index("s")
        wid = s * NC + c
        base = wid * E_PER_W

        pltpu.sync_copy(zeros_hbm, acc.at[pl.ds(s * ROWS_PER_TILE, ROWS_PER_TILE)])

        @pl.when(s == NS - 1)
        def _():
            pltpu.sync_copy(zeros_hbm.at[pl.ds(0, TAIL_ROWS)],
                            acc.at[pl.ds(TAIL_OFF, TAIL_ROWS)])

        plsc.subcore_barrier()

        pltpu.sync_copy(src_hbm.at[pl.ds(base, CHUNK)], src0)
        pltpu.sync_copy(dst_hbm.at[pl.ds(base, CHUNK)], dst0)
        pltpu.async_copy(feat_hbm.at[src0], rows0, sem0)

        def body(g, _):
            j0 = 2 * g
            off1 = base + (j0 + 1) * CHUNK
            pltpu.sync_copy(src_hbm.at[pl.ds(off1, CHUNK)], src1)
            pltpu.sync_copy(dst_hbm.at[pl.ds(off1, CHUNK)], dst1)
            pltpu.async_copy(feat_hbm.at[src1], rows1, sem1)
            pltpu.make_async_copy(feat_hbm.at[src0], rows0, sem0).wait()
            pltpu.sync_copy(rows0, acc.at[dst0], add=True)

            @pl.when(j0 + 2 < FULL_CHUNKS)
            def _():
                off2 = base + (j0 + 2) * CHUNK
                pltpu.sync_copy(src_hbm.at[pl.ds(off2, CHUNK)], src0)
                pltpu.sync_copy(dst_hbm.at[pl.ds(off2, CHUNK)], dst0)
                pltpu.async_copy(feat_hbm.at[src0], rows0, sem0)

            pltpu.make_async_copy(feat_hbm.at[src1], rows1, sem1).wait()
            pltpu.sync_copy(rows1, acc.at[dst1], add=True)
            return ()

        lax.fori_loop(0, FULL_CHUNKS // 2, body, ())

        toff = base + FULL_CHUNKS * CHUNK
        pltpu.sync_copy(src_hbm.at[pl.ds(toff, TAIL_E)], srct)
        pltpu.sync_copy(dst_hbm.at[pl.ds(toff, TAIL_E)], dstt)
        pltpu.async_copy(feat_hbm.at[srct], rowst, sem0).wait()
        pltpu.sync_copy(rowst, acc.at[dstt], add=True)

        plsc.subcore_barrier()
        pltpu.sync_copy(
            acc.at[pl.ds(s * ROWS_PER_TILE, ROWS_PER_TILE)],
            out_hbm.at[c, pl.ds(s * ROWS_PER_TILE, ROWS_PER_TILE)],
        )

        @pl.when(s == NS - 1)
        def _():
            pltpu.sync_copy(acc.at[pl.ds(TAIL_OFF, TAIL_ROWS)],
                            out_hbm.at[c, pl.ds(TAIL_OFF, TAIL_ROWS)])

    return k(features, src, dst, zeros)


def _tc_body(f_ref, p0_ref, p1_ref, wt_ref, b_ref, t_ref, o_ref):
    t1 = t_ref[0, 0]
    t2 = t_ref[0, 1]
    wt = wt_ref[...]
    b = b_ref[...]
    a1 = jnp.dot(f_ref[...], wt, preferred_element_type=jnp.float32) + b
    hn = p0_ref[...] + p1_ref[...]
    a2 = jnp.dot(hn, wt, preferred_element_type=jnp.float32) + b
    o_ref[...] = t1 * jnp.maximum(a1, 0.0) + t2 * jnp.maximum(a2, 0.0)


def _tc_combine(features, partials, W, b, theta1, theta2):
    wt = W.T
    b2 = b.reshape(1, D)
    thetas = jnp.stack([theta1[0], theta2[0]]).reshape(1, 2)
    R = 1000
    grid = (N_NODES // R,)
    return pl.pallas_call(
        _tc_body,
        grid=grid,
        in_specs=[
            pl.BlockSpec((R, D), lambda i: (i, 0)),
            pl.BlockSpec((R, D), lambda i: (i, 0)),
            pl.BlockSpec((R, D), lambda i: (i, 0)),
            pl.BlockSpec((D, D), lambda i: (0, 0)),
            pl.BlockSpec((1, D), lambda i: (0, 0)),
            pl.BlockSpec(memory_space=pltpu.SMEM),
        ],
        out_specs=pl.BlockSpec((R, D), lambda i: (i, 0)),
        out_shape=jax.ShapeDtypeStruct((N_NODES, D), jnp.float32),
    )(features, partials[0], partials[1], wt, b2, thetas)


@jax.jit
def kernel(features, edge_index, W, b, theta1, theta2):
    src = edge_index[0].astype(jnp.int32)
    dst = edge_index[1].astype(jnp.int32)
    zeros = jnp.zeros((ROWS_PER_TILE, D), jnp.float32)
    partials = _sc_scatter_sum(features, src, dst, zeros)
    return _tc_combine(features, partials, W, b, theta1, theta2)
